# Initial kernel scaffold; baseline (speedup 1.0000x reference)
import jax
import jax.numpy as jnp
from jax.experimental import pallas as pl


def kernel(tokenized_prompts, token_embedding, ctx, proj_W, proj_b, cp0, cp1, pw0, pb0, pw1, pb1):
    raise NotImplementedError("write your pallas kernel here")


# speedup vs baseline: 1.8370x; 1.8370x over previous
import functools

import jax
import jax.numpy as jnp
from jax import lax
from jax.experimental import pallas as pl
from jax.experimental.pallas import tpu as pltpu
from jax.experimental.pallas import tpu_sc as plsc


def _sc_gather_fn(n_rows, d, n_workers, nc, chunk, seq, n_ctx):
    b_per_w = n_rows // n_workers
    n_chunks = b_per_w // chunk
    assert n_chunks % 2 == 0 and n_chunks * chunk == b_per_w
    assert b_per_w % seq == 0

    mesh = plsc.VectorSubcoreMesh(core_axis_name="c", subcore_axis_name="s")

    @functools.partial(
        pl.kernel,
        out_type=jax.ShapeDtypeStruct((n_rows, d), jnp.float32),
        mesh=mesh,
        scratch_types=[
            pltpu.VMEM((b_per_w,), jnp.int32),
            pltpu.VMEM((2, chunk, d), jnp.float32),
            pltpu.VMEM_SHARED((n_ctx, d), jnp.float32),
            pltpu.SemaphoreType.DMA,
            pltpu.SemaphoreType.DMA,
            pltpu.SemaphoreType.DMA,
            pltpu.SemaphoreType.DMA,
        ],
    )
    def k(table_hbm, idx_hbm, ctx_hbm, out_hbm, idx_v, rows_v, ctx_sh,
          gsem0, gsem1, wsem0, wsem1):
        gsem = (gsem0, gsem1)
        wsem = (wsem0, wsem1)
        sid = lax.axis_index("s")
        wid = sid * nc + lax.axis_index("c")
        base = wid * b_per_w

        @pl.when(sid == 0)
        def _():
            pltpu.sync_copy(ctx_hbm, ctx_sh)

        pltpu.sync_copy(idx_hbm.at[pl.ds(base, b_per_w)], idx_v)
        plsc.subcore_barrier()

        def start_gather(g, b):
            pltpu.async_copy(
                table_hbm.at[idx_v.at[pl.ds(g * chunk, chunk)]],
                rows_v.at[b], gsem[b])

        def wait_gather(b):
            pltpu.make_async_copy(
                table_hbm.at[idx_v.at[pl.ds(0, chunk)]],
                rows_v.at[b], gsem[b]).wait()

        def start_write(g, b):
            pltpu.async_copy(
                rows_v.at[b], out_hbm.at[pl.ds(base + g * chunk, chunk)],
                wsem[b])

        def wait_write(b):
            pltpu.make_async_copy(
                rows_v.at[b], out_hbm.at[pl.ds(0, chunk)], wsem[b]).wait()

        start_gather(0, 0)
        start_gather(1, 1)

        def patch_ctx(g, b):
            lrow0 = g * chunk
            cc0 = lax.div(lrow0, seq)
            for co in range(chunk // seq + 2):
                for p in range(n_ctx):
                    lr = (cc0 + co) * seq + (1 + p) - lrow0

                    @pl.when(jnp.logical_and(lr >= 0, lr < chunk))
                    def _(lr=lr, p=p, b=b):
                        pltpu.sync_copy(ctx_sh.at[pl.ds(p, 1)],
                                        rows_v.at[b].at[pl.ds(lr, 1)])

        def pair_body(j, carry):
            for b in range(2):
                g = 2 * j + b
                wait_gather(b)
                patch_ctx(g, b)
                start_write(g, b)

                @pl.when(g + 2 < n_chunks)
                def _():
                    wait_write(b)
                    start_gather(g + 2, b)
            return carry

        lax.fori_loop(0, n_chunks // 2, pair_body, 0)
        wait_write(0)
        wait_write(1)

    return k


def _mm_body(c_ref, w_ref, b_ref, c0_ref, w0_ref, b0_ref, c1_ref, w1_ref,
             b1_ref, o_ref, o0_ref, o1_ref):
    o_ref[...] = jnp.dot(c_ref[...], w_ref[...],
                         preferred_element_type=jnp.float32) + b_ref[...]
    o0_ref[...] = jnp.dot(c0_ref[...], w0_ref[...],
                          preferred_element_type=jnp.float32) + b0_ref[...]
    o1_ref[...] = jnp.dot(c1_ref[...], w1_ref[...],
                          preferred_element_type=jnp.float32) + b1_ref[...]


def kernel(tokenized_prompts, token_embedding, ctx, proj_W, proj_b, cp0, cp1,
           pw0, pb0, pw1, pb1):
    n_cls, seq = tokenized_prompts.shape
    vocab, d = token_embedding.shape
    pd = proj_W.shape[1]
    n_rows = n_cls * seq

    info = plsc.get_sparse_core_info()
    nc, ns = info.num_cores, info.num_subcores
    nw = nc * ns

    idx_flat = tokenized_prompts.reshape(n_rows)
    prompts_flat = _sc_gather_fn(n_rows, d, nw, nc, 112, seq, ctx.shape[0])(
        token_embedding, idx_flat, ctx)
    prompts = prompts_flat.reshape(n_cls, seq, d)

    n_ctx = ctx.shape[0]
    out_sds = jax.ShapeDtypeStruct((n_ctx, pd), jnp.float32)
    proj_ctx, vd0, vd1 = pl.pallas_call(
        _mm_body,
        out_shape=[out_sds, out_sds, out_sds],
    )(ctx, proj_W, proj_b.reshape(1, pd), cp0, pw0, pb0.reshape(1, pd),
      cp1, pw1, pb1.reshape(1, pd))

    return (prompts, tokenized_prompts, proj_ctx, cp0, cp1, vd0, vd1)


# speedup vs baseline: 4.9790x; 2.7103x over previous
import functools

import jax
import jax.numpy as jnp
from jax import lax
from jax.experimental import pallas as pl
from jax.experimental.pallas import tpu as pltpu
from jax.experimental.pallas import tpu_sc as plsc


def _sc_gather_fn(n_cls, seq, d, n_workers, nc, n_ctx, chunk):
    n_rows = n_cls * seq
    head = n_cls
    ctx_lo = n_ctx * n_cls
    tail0 = head + ctx_lo
    tail = n_rows - tail0
    tail_w = tail // n_workers
    head_w = head // n_workers
    fill_w = ctx_lo // n_workers
    n_chunks = tail_w // chunk
    assert n_chunks * chunk == tail_w and head_w * n_workers == head
    assert fill_w * n_workers == ctx_lo and fill_w % 8 == 0
    assert n_cls % fill_w == 0
    assert chunk % 8 == 0 and head_w % 8 == 0 and tail_w % 8 == 0

    mesh = plsc.VectorSubcoreMesh(core_axis_name="c", subcore_axis_name="s")

    @functools.partial(
        pl.kernel,
        out_type=jax.ShapeDtypeStruct((n_rows, d), jnp.float32),
        mesh=mesh,
        scratch_types=[
            pltpu.VMEM((tail_w + head_w,), jnp.int32),
            pltpu.VMEM((2, chunk, d), jnp.float32),
            pltpu.VMEM((fill_w, d), jnp.float32),
            pltpu.VMEM((n_ctx, d), jnp.float32),
            pltpu.SemaphoreType.DMA,
            pltpu.SemaphoreType.DMA,
            pltpu.SemaphoreType.DMA,
            pltpu.SemaphoreType.DMA,
        ],
    )
    def k(table_hbm, idx_hbm, ctx_hbm, out_hbm, idx_v, rows_v, rep_v, ctx_v,
          gsem0, gsem1, wsem0, wsem1):
        gsem = (gsem0, gsem1)
        wsem = (wsem0, wsem1)
        wid = lax.axis_index("s") * nc + lax.axis_index("c")
        tbase = tail0 + wid * tail_w

        pltpu.sync_copy(idx_hbm.at[pl.ds(tbase, tail_w)],
                        idx_v.at[pl.ds(0, tail_w)])
        pltpu.sync_copy(idx_hbm.at[pl.ds(wid * head_w, head_w)],
                        idx_v.at[pl.ds(tail_w, head_w)])
        pltpu.sync_copy(ctx_hbm, ctx_v)

        pltpu.async_copy(
            table_hbm.at[idx_v.at[pl.ds(tail_w, head_w)]],
            rows_v.at[0].at[pl.ds(0, head_w)], gsem0).wait()
        pltpu.async_copy(
            rows_v.at[0].at[pl.ds(0, head_w)],
            out_hbm.at[pl.ds(wid * head_w, head_w)], wsem0).wait()

        sel = lax.div(wid * fill_w, n_cls)

        def fill_row(i, carry):
            for j in range(d // 16):
                rep_v[i, pl.ds(16 * j, 16)] = ctx_v[sel, pl.ds(16 * j, 16)]
            return carry

        lax.fori_loop(0, fill_w, fill_row, 0)
        pltpu.async_copy(
            rep_v, out_hbm.at[pl.ds(head + wid * fill_w, fill_w)],
            wsem0).wait()

        def start_gather(g, b):
            pltpu.async_copy(
                table_hbm.at[idx_v.at[pl.ds(g * chunk, chunk)]],
                rows_v.at[b], gsem[b])

        def wait_gather(b):
            pltpu.make_async_copy(
                table_hbm.at[idx_v.at[pl.ds(0, chunk)]],
                rows_v.at[b], gsem[b]).wait()

        def start_write(g, b):
            pltpu.async_copy(
                rows_v.at[b], out_hbm.at[pl.ds(tbase + g * chunk, chunk)],
                wsem[b])

        def wait_write(b):
            pltpu.make_async_copy(
                rows_v.at[b], out_hbm.at[pl.ds(0, chunk)], wsem[b]).wait()

        start_gather(0, 0)
        start_gather(1, 1)

        def pair_body(j, carry):
            for b in range(2):
                g = 2 * j + b

                @pl.when(g < n_chunks)
                def _(g=g, b=b):
                    wait_gather(b)
                    start_write(g, b)

                    @pl.when(g + 2 < n_chunks)
                    def _():
                        wait_write(b)
                        start_gather(g + 2, b)
            return carry

        lax.fori_loop(0, (n_chunks + 1) // 2, pair_body, 0)
        wait_write(0)
        wait_write(1)

    return k


def _mm_body(c_ref, w_ref, b_ref, c0_ref, w0_ref, b0_ref, c1_ref, w1_ref,
             b1_ref, o_ref, o0_ref, o1_ref):
    o_ref[...] = jnp.dot(c_ref[...], w_ref[...],
                         preferred_element_type=jnp.float32) + b_ref[...]
    o0_ref[...] = jnp.dot(c0_ref[...], w0_ref[...],
                          preferred_element_type=jnp.float32) + b0_ref[...]
    o1_ref[...] = jnp.dot(c1_ref[...], w1_ref[...],
                          preferred_element_type=jnp.float32) + b1_ref[...]


def kernel(tokenized_prompts, token_embedding, ctx, proj_W, proj_b, cp0, cp1,
           pw0, pb0, pw1, pb1):
    n_cls, seq = tokenized_prompts.shape
    vocab, d = token_embedding.shape
    pd = proj_W.shape[1]
    n_ctx = ctx.shape[0]

    info = plsc.get_sparse_core_info()
    nc, ns = info.num_cores, info.num_subcores
    nw = nc * ns

    idx_t = tokenized_prompts.T.reshape(seq * n_cls)
    flat = _sc_gather_fn(n_cls, seq, d, nw, nc, n_ctx, 64)(
        token_embedding, idx_t, ctx)
    prompts = flat.reshape(seq, n_cls, d).transpose(1, 0, 2)

    out_sds = jax.ShapeDtypeStruct((n_ctx, pd), jnp.float32)
    proj_ctx, vd0, vd1 = pl.pallas_call(
        _mm_body,
        out_shape=[out_sds, out_sds, out_sds],
    )(ctx, proj_W, proj_b.reshape(1, pd), cp0, pw0, pb0.reshape(1, pd),
      cp1, pw1, pb1.reshape(1, pd))

    return (prompts, tokenized_prompts, proj_ctx, cp0, cp1, vd0, vd1)


# speedup vs baseline: 5.0843x; 1.0211x over previous
import functools

import jax
import jax.numpy as jnp
from jax import lax
from jax.experimental import pallas as pl
from jax.experimental.pallas import tpu as pltpu
from jax.experimental.pallas import tpu_sc as plsc


def _sc_gather_fn(n_cls, seq, d, n_workers, nc, n_ctx, chunk):
    n_rows = n_cls * seq
    head = n_cls
    ctx_lo = n_ctx * n_cls
    tail0 = head + ctx_lo
    tail = n_rows - tail0
    tail_w = tail // n_workers
    head_w = head // n_workers
    fill_w = ctx_lo // n_workers
    n_chunks = tail_w // chunk
    rem = tail_w - n_chunks * chunk
    assert tail_w * n_workers == tail and head_w * n_workers == head
    assert fill_w * n_workers == ctx_lo and fill_w % 8 == 0
    assert n_cls % fill_w == 0
    assert chunk % 8 == 0 and head_w % 8 == 0 and tail_w % 8 == 0
    assert rem % 8 == 0 and rem + head_w <= chunk and fill_w <= chunk

    mesh = plsc.VectorSubcoreMesh(core_axis_name="c", subcore_axis_name="s")

    @functools.partial(
        pl.kernel,
        out_type=jax.ShapeDtypeStruct((n_rows, d), jnp.float32),
        mesh=mesh,
        scratch_types=[
            pltpu.VMEM((tail_w + head_w,), jnp.int32),
            pltpu.VMEM((2, chunk, d), jnp.float32),
            pltpu.VMEM((n_ctx, d), jnp.float32),
            pltpu.SemaphoreType.DMA,
            pltpu.SemaphoreType.DMA,
            pltpu.SemaphoreType.DMA,
            pltpu.SemaphoreType.DMA,
        ],
    )
    def k(table_hbm, idx_hbm, ctx_hbm, out_hbm, idx_v, rows_v, ctx_v,
          gsem0, gsem1, wsem0, wsem1):
        gsem = (gsem0, gsem1)
        wsem = (wsem0, wsem1)
        wid = lax.axis_index("s") * nc + lax.axis_index("c")
        tbase = tail0 + wid * tail_w

        pltpu.sync_copy(idx_hbm.at[pl.ds(tbase, tail_w)],
                        idx_v.at[pl.ds(0, tail_w)])

        def start_gather(g, b):
            pltpu.async_copy(
                table_hbm.at[idx_v.at[pl.ds(g * chunk, chunk)]],
                rows_v.at[b], gsem[b])

        def wait_gather(b):
            pltpu.make_async_copy(
                table_hbm.at[idx_v.at[pl.ds(0, chunk)]],
                rows_v.at[b], gsem[b]).wait()

        def start_write(g, b):
            pltpu.async_copy(
                rows_v.at[b], out_hbm.at[pl.ds(tbase + g * chunk, chunk)],
                wsem[b])

        def wait_write(b):
            pltpu.make_async_copy(
                rows_v.at[b], out_hbm.at[pl.ds(0, chunk)], wsem[b]).wait()

        start_gather(0, 0)
        start_gather(1, 1)

        def pair_body(j, carry):
            for b in range(2):
                g = 2 * j + b

                @pl.when(g < n_chunks)
                def _(g=g, b=b):
                    wait_gather(b)
                    start_write(g, b)

                    @pl.when(g + 2 < n_chunks)
                    def _():
                        wait_write(b)
                        start_gather(g + 2, b)
            return carry

        lax.fori_loop(0, (n_chunks + 1) // 2, pair_body, 0)
        wait_write(0)
        wait_write(1)

        pltpu.sync_copy(idx_hbm.at[pl.ds(wid * head_w, head_w)],
                        idx_v.at[pl.ds(tail_w, head_w)])
        pltpu.sync_copy(ctx_hbm, ctx_v)
        pltpu.async_copy(
            table_hbm.at[idx_v.at[pl.ds(n_chunks * chunk, rem + head_w)]],
            rows_v.at[1].at[pl.ds(0, rem + head_w)], gsem1)

        sel = lax.div(wid * fill_w, n_cls)

        def fill_row(i, carry):
            for j in range(d // 16):
                rows_v[0, i, pl.ds(16 * j, 16)] = ctx_v[sel, pl.ds(16 * j, 16)]
            return carry

        lax.fori_loop(0, fill_w, fill_row, 0)
        pltpu.async_copy(
            rows_v.at[0].at[pl.ds(0, fill_w)],
            out_hbm.at[pl.ds(head + wid * fill_w, fill_w)], wsem0)

        pltpu.make_async_copy(
            table_hbm.at[idx_v.at[pl.ds(0, rem + head_w)]],
            rows_v.at[1].at[pl.ds(0, rem + head_w)], gsem1).wait()
        if rem:
            pltpu.async_copy(
                rows_v.at[1].at[pl.ds(0, rem)],
                out_hbm.at[pl.ds(tbase + n_chunks * chunk, rem)], wsem1)
        pltpu.async_copy(
            rows_v.at[1].at[pl.ds(rem, head_w)],
            out_hbm.at[pl.ds(wid * head_w, head_w)], wsem1)

        pltpu.make_async_copy(
            rows_v.at[0].at[pl.ds(0, fill_w)],
            out_hbm.at[pl.ds(0, fill_w)], wsem0).wait()
        if rem:
            pltpu.make_async_copy(
                rows_v.at[1].at[pl.ds(0, rem)],
                out_hbm.at[pl.ds(0, rem)], wsem1).wait()
        pltpu.make_async_copy(
            rows_v.at[1].at[pl.ds(0, head_w)],
            out_hbm.at[pl.ds(0, head_w)], wsem1).wait()

    return k


def _mm_body(c_ref, w_ref, b_ref, c0_ref, w0_ref, b0_ref, c1_ref, w1_ref,
             b1_ref, o_ref, o0_ref, o1_ref):
    o_ref[...] = jnp.dot(c_ref[...], w_ref[...],
                         preferred_element_type=jnp.float32) + b_ref[...]
    o0_ref[...] = jnp.dot(c0_ref[...], w0_ref[...],
                          preferred_element_type=jnp.float32) + b0_ref[...]
    o1_ref[...] = jnp.dot(c1_ref[...], w1_ref[...],
                          preferred_element_type=jnp.float32) + b1_ref[...]


def kernel(tokenized_prompts, token_embedding, ctx, proj_W, proj_b, cp0, cp1,
           pw0, pb0, pw1, pb1):
    n_cls, seq = tokenized_prompts.shape
    vocab, d = token_embedding.shape
    pd = proj_W.shape[1]
    n_ctx = ctx.shape[0]

    info = plsc.get_sparse_core_info()
    nc, ns = info.num_cores, info.num_subcores
    nw = nc * ns

    idx_t = tokenized_prompts.T.reshape(seq * n_cls)
    flat = _sc_gather_fn(n_cls, seq, d, nw, nc, n_ctx, 112)(
        token_embedding, idx_t, ctx)
    prompts = flat.reshape(seq, n_cls, d).transpose(1, 0, 2)

    out_sds = jax.ShapeDtypeStruct((n_ctx, pd), jnp.float32)
    proj_ctx, vd0, vd1 = pl.pallas_call(
        _mm_body,
        out_shape=[out_sds, out_sds, out_sds],
    )(ctx, proj_W, proj_b.reshape(1, pd), cp0, pw0, pb0.reshape(1, pd),
      cp1, pw1, pb1.reshape(1, pd))

    return (prompts, tokenized_prompts, proj_ctx, cp0, cp1, vd0, vd1)
